# baseline (speedup 1.0000x reference)
import jax
import jax.numpy as jnp
from jax.experimental import pallas as pl
from jax.experimental.pallas import tpu as pltpu

P = 12
Q = 12
D = 64
N = 207
B = 16
NP = 256
M = 5
F32 = jnp.float32
BF16 = jnp.bfloat16


def _dcgru_kernel(Xp_r, tod_r, adj_r, adjT_r,
                  W1x_r, W1t_r, b1_r, W2T_r, b2_r,
                  WruTe_r, brue_r, WcTe_r, bce_r,
                  WruTd_r, brud_r, WcTd_r, bcd_r,
                  Wo1T_r, bo1_r, w2c_r, bo2_r,
                  out_r, h_r):
    dot = lambda a, b: jnp.dot(a, b, preferred_element_type=F32)

    A = adj_r[...]
    d1 = jnp.sum(A, axis=1, keepdims=True)
    R1 = (jnp.where(d1 > 0, 1.0 / d1, 0.0) * A).astype(BF16)
    AT = adjT_r[...]
    d2 = jnp.sum(AT, axis=1, keepdims=True)
    R2 = (jnp.where(d2 > 0, 1.0 / d2, 0.0) * AT).astype(BF16)
    R12 = jnp.concatenate([R1, R2], axis=1)

    h_r[...] = jnp.zeros((B * D, NP), F32)

    def dconv(xh, tb, WT, bcol):
        m13 = dot(xh, R12)
        m1b = m13[:, :NP].astype(BF16)
        m3b = m13[:, NP:].astype(BF16)
        xf = xh.astype(F32)
        m2b = (2.0 * dot(m1b, R1) - xf).astype(BF16)
        m4b = (2.0 * dot(m3b, R2) - xf).astype(BF16)
        mats = (xh, m1b, m2b, m3b, m4b)
        outs = []
        for b in range(B):
            Zb = jnp.concatenate(
                [m[b * tb:(b + 1) * tb, :] for m in mats], axis=0)
            outs.append(dot(WT, Zb) + bcol)
        return outs

    W1x = W1x_r[...]
    W1t = W1t_r[...]
    b1 = b1_r[...]
    W2T = W2T_r[...]
    b2 = b2_r[...]
    WruTe = WruTe_r[...]
    brue = brue_r[...]
    WcTe = WcTe_r[...]
    bce = bce_r[...]

    def enc_body(p, carry):
        h = h_r[...]
        xr = Xp_r[p]
        tr = tod_r[p]
        xs = []
        hs = []
        pieces = []
        for b in range(B):
            arg = W1x * xr[b:b + 1, :] + W1t * tr[b:b + 1, :] + b1
            xb = (dot(W2T, jnp.maximum(arg, 0.0).astype(BF16)) +
                  b2).astype(BF16)
            xs.append(xb)
            hs.append(h[b * D:(b + 1) * D, :])
            pieces.append(xb)
            pieces.append(hs[b].astype(BF16))
        xh = jnp.concatenate(pieces, 0)
        ru = dconv(xh, 2 * D, WruTe, brue)
        pieces2 = []
        us = []
        for b in range(B):
            r_b = jax.nn.sigmoid(ru[b][:D])
            us.append(jax.nn.sigmoid(ru[b][D:]))
            pieces2.append(xs[b])
            pieces2.append((r_b * hs[b]).astype(BF16))
        xh2 = jnp.concatenate(pieces2, 0)
        cs = dconv(xh2, 2 * D, WcTe, bce)
        hn = [us[b] * hs[b] + (1.0 - us[b]) * jnp.tanh(cs[b])
              for b in range(B)]
        h_r[...] = jnp.concatenate(hn, 0)
        return carry

    jax.lax.fori_loop(0, P, enc_body, 0)

    WruTd = WruTd_r[...]
    brud = brud_r[...]
    WcTd = WcTd_r[...]
    bcd = bcd_r[...]
    Wo1T = Wo1T_r[...]
    bo1 = bo1_r[...]
    w2c = w2c_r[...]
    bo2 = bo2_r[...]

    def dec_body(q, carry):
        h = h_r[...]
        hb = h.astype(BF16)
        hs = [h[b * D:(b + 1) * D, :] for b in range(B)]
        ru = dconv(hb, D, WruTd, brud)
        pieces = []
        us = []
        for b in range(B):
            r_b = jax.nn.sigmoid(ru[b][:D])
            us.append(jax.nn.sigmoid(ru[b][D:]))
            pieces.append((r_b * hs[b]).astype(BF16))
        rh = jnp.concatenate(pieces, 0)
        cs = dconv(rh, D, WcTd, bcd)
        hn = []
        srows = []
        for b in range(B):
            h2b = us[b] * hs[b] + (1.0 - us[b]) * jnp.tanh(cs[b])
            hn.append(h2b)
            o1 = jnp.maximum(dot(Wo1T, h2b.astype(BF16)) + bo1, 0.0)
            srows.append(jnp.sum(o1 * w2c, axis=0, keepdims=True))
        h_r[...] = jnp.concatenate(hn, 0)
        out_r[q] = jnp.concatenate(srows, 0) + bo2
        return carry

    jax.lax.fori_loop(0, Q, dec_body, 0)


def _dcgru_call(Xp, todp, adjp, adjTp, W1x, W1t, b1, W2T, b2,
                WruTe, brue, WcTe, bce, WruTd, brud, WcTd, bcd,
                Wo1T, bo1, w2c, bo2, interpret=False):
    return pl.pallas_call(
        _dcgru_kernel,
        out_shape=jax.ShapeDtypeStruct((Q, B, NP), F32),
        scratch_shapes=[pltpu.VMEM((B * D, NP), F32)],
        interpret=interpret,
    )(Xp, todp, adjp, adjTp, W1x, W1t, b1, W2T, b2,
      WruTe, brue, WcTe, bce, WruTd, brud, WcTd, bcd,
      Wo1T, bo1, w2c, bo2)


def kernel(X, TE, adj_mx, W_in1, b_in1, W_in2, b_in2,
           enc_W_ru, enc_b_ru, enc_W_c, enc_b_c,
           dec_W_ru, dec_b_ru, dec_W_c, dec_b_c,
           W_out1, b_out1, W_out2, b_out2):
    f32 = F32
    ts = 2 * D

    Xsq = X[..., 0].astype(f32)
    Xp = jnp.zeros((P, B, NP), f32).at[:, :, :N].set(Xsq.transpose(1, 0, 2))
    tod = TE[:, :P, -1].astype(f32) / (12.0 * 24.0)
    todp = jnp.broadcast_to(tod.T[:, :, None], (P, B, NP))

    adjp = jnp.zeros((NP, NP), f32).at[:N, :N].set(adj_mx)
    adjTp = jnp.zeros((NP, NP), f32).at[:N, :N].set(adj_mx.T)

    W1x = W_in1[0][:, None]
    W1t = W_in1[1][:, None]
    b1 = b_in1[:, None]
    W2T = W_in2.T.astype(BF16)
    b2 = b_in2[:, None]

    WruTe = enc_W_ru.reshape(ts, M, ts).transpose(2, 1, 0).reshape(
        ts, M * ts).astype(BF16)
    brue = enc_b_ru[:, None]
    WcTe = enc_W_c.reshape(ts, M, D).transpose(2, 1, 0).reshape(
        D, M * ts).astype(BF16)
    bce = enc_b_c[:, None]
    WruTd = dec_W_ru.reshape(ts, M, ts)[D:].transpose(2, 1, 0).reshape(
        ts, M * D).astype(BF16)
    brud = dec_b_ru[:, None]
    WcTd = dec_W_c.reshape(ts, M, D)[D:].transpose(2, 1, 0).reshape(
        D, M * D).astype(BF16)
    bcd = dec_b_c[:, None]

    Wo1T = W_out1.T.astype(BF16)
    bo1 = b_out1[:, None]
    w2c = W_out2[:, 0][:, None]
    bo2 = b_out2.reshape(1, 1)

    out = _dcgru_call(Xp, todp, adjp, adjTp, W1x, W1t, b1, W2T, b2,
                      WruTe, brue, WcTe, bce, WruTd, brud, WcTd, bcd,
                      Wo1T, bo1, w2c, bo2)
    return out.transpose(1, 0, 2)[:, :, :N, None]


# speedup vs baseline: 1.0601x; 1.0601x over previous
import jax
import jax.numpy as jnp
from jax.experimental import pallas as pl
from jax.experimental.pallas import tpu as pltpu

P = 12
Q = 12
D = 64
N = 207
B = 16
NP = 256
M = 5
F32 = jnp.float32
BF16 = jnp.bfloat16


def _dcgru_kernel(Xp_r, tod_r, adj_r, adjT_r,
                  W1x_r, W1t_r, b1_r, W2T_r, b2_r,
                  WruTe_r, brue_r, WcTe_r, bce_r,
                  WruTd_r, brud_r, WcTd_r, bcd_r,
                  Wo1T_r, bo1_r, w2c_r, bo2_r,
                  out_r, h_r):
    dot = lambda a, b: jnp.dot(a, b, preferred_element_type=F32)

    A = adj_r[...]
    d1 = jnp.sum(A, axis=1, keepdims=True)
    R1 = (jnp.where(d1 > 0, 1.0 / d1, 0.0) * A).astype(BF16)
    AT = adjT_r[...]
    d2 = jnp.sum(AT, axis=1, keepdims=True)
    R2 = (jnp.where(d2 > 0, 1.0 / d2, 0.0) * AT).astype(BF16)
    eye = (jax.lax.broadcasted_iota(jnp.int32, (NP, NP), 0) ==
           jax.lax.broadcasted_iota(jnp.int32, (NP, NP), 1)).astype(F32)
    P2 = (2.0 * dot(R1, R1) - eye).astype(BF16)
    P4 = (2.0 * dot(R2, R2) - eye).astype(BF16)
    RP = jnp.concatenate([R1, P2, R2, P4], axis=1)

    h_r[...] = jnp.zeros((B * D, NP), F32)

    def diffuse(v):
        mm = dot(v, RP).astype(BF16)
        return (v, mm[:, :NP], mm[:, NP:2 * NP],
                mm[:, 2 * NP:3 * NP], mm[:, 3 * NP:])

    def project(mats, WT, bcol):
        outs = []
        for b in range(B):
            pieces = []
            for k in range(5):
                for g, tg in mats:
                    pieces.append(g[k][b * tg:(b + 1) * tg, :])
            Zb = jnp.concatenate(pieces, axis=0)
            outs.append(dot(WT, Zb) + bcol)
        return outs

    W1x = W1x_r[...]
    W1t = W1t_r[...]
    b1 = b1_r[...]
    W2T = W2T_r[...]
    b2 = b2_r[...]
    WruTe = WruTe_r[...]
    brue = brue_r[...]
    WcTe = WcTe_r[...]
    bce = bce_r[...]

    def enc_body(p, carry):
        h = h_r[...]
        hb = h.astype(BF16)
        hs = [h[b * D:(b + 1) * D, :] for b in range(B)]
        xr = Xp_r[p]
        tr = tod_r[p]
        xs = []
        for b in range(B):
            arg = W1x * xr[b:b + 1, :] + W1t * tr[b:b + 1, :] + b1
            xs.append((dot(W2T, jnp.maximum(arg, 0.0).astype(BF16)) +
                       b2).astype(BF16))
        xcat = jnp.concatenate(xs, 0)
        xm = diffuse(xcat)
        hm = diffuse(hb)
        ru = project([(xm, D), (hm, D)], WruTe, brue)
        pieces = []
        us = []
        for b in range(B):
            r_b = jax.nn.sigmoid(ru[b][:D])
            us.append(jax.nn.sigmoid(ru[b][D:]))
            pieces.append((r_b * hs[b]).astype(BF16))
        rh = jnp.concatenate(pieces, 0)
        rm = diffuse(rh)
        cs = project([(xm, D), (rm, D)], WcTe, bce)
        hn = [us[b] * hs[b] + (1.0 - us[b]) * jnp.tanh(cs[b])
              for b in range(B)]
        h_r[...] = jnp.concatenate(hn, 0)
        return carry

    jax.lax.fori_loop(0, P, enc_body, 0)

    WruTd = WruTd_r[...]
    brud = brud_r[...]
    WcTd = WcTd_r[...]
    bcd = bcd_r[...]
    Wo1T = Wo1T_r[...]
    bo1 = bo1_r[...]
    w2c = w2c_r[...]
    bo2 = bo2_r[...]

    def dec_body(q, carry):
        h = h_r[...]
        hb = h.astype(BF16)
        hs = [h[b * D:(b + 1) * D, :] for b in range(B)]
        hm = diffuse(hb)
        ru = project([(hm, D)], WruTd, brud)
        pieces = []
        us = []
        for b in range(B):
            r_b = jax.nn.sigmoid(ru[b][:D])
            us.append(jax.nn.sigmoid(ru[b][D:]))
            pieces.append((r_b * hs[b]).astype(BF16))
        rh = jnp.concatenate(pieces, 0)
        rm = diffuse(rh)
        cs = project([(rm, D)], WcTd, bcd)
        hn = []
        srows = []
        for b in range(B):
            h2b = us[b] * hs[b] + (1.0 - us[b]) * jnp.tanh(cs[b])
            hn.append(h2b)
            o1 = jnp.maximum(dot(Wo1T, h2b.astype(BF16)) + bo1, 0.0)
            srows.append(jnp.sum(o1 * w2c, axis=0, keepdims=True))
        h_r[...] = jnp.concatenate(hn, 0)
        out_r[q] = jnp.concatenate(srows, 0) + bo2
        return carry

    jax.lax.fori_loop(0, Q, dec_body, 0)


def _dcgru_call(Xp, todp, adjp, adjTp, W1x, W1t, b1, W2T, b2,
                WruTe, brue, WcTe, bce, WruTd, brud, WcTd, bcd,
                Wo1T, bo1, w2c, bo2, interpret=False):
    return pl.pallas_call(
        _dcgru_kernel,
        out_shape=jax.ShapeDtypeStruct((Q, B, NP), F32),
        scratch_shapes=[pltpu.VMEM((B * D, NP), F32)],
        interpret=interpret,
    )(Xp, todp, adjp, adjTp, W1x, W1t, b1, W2T, b2,
      WruTe, brue, WcTe, bce, WruTd, brud, WcTd, bcd,
      Wo1T, bo1, w2c, bo2)


def kernel(X, TE, adj_mx, W_in1, b_in1, W_in2, b_in2,
           enc_W_ru, enc_b_ru, enc_W_c, enc_b_c,
           dec_W_ru, dec_b_ru, dec_W_c, dec_b_c,
           W_out1, b_out1, W_out2, b_out2):
    f32 = F32
    ts = 2 * D

    Xsq = X[..., 0].astype(f32)
    Xp = jnp.zeros((P, B, NP), f32).at[:, :, :N].set(Xsq.transpose(1, 0, 2))
    tod = TE[:, :P, -1].astype(f32) / (12.0 * 24.0)
    todp = jnp.broadcast_to(tod.T[:, :, None], (P, B, NP))

    adjp = jnp.zeros((NP, NP), f32).at[:N, :N].set(adj_mx)
    adjTp = jnp.zeros((NP, NP), f32).at[:N, :N].set(adj_mx.T)

    W1x = W_in1[0][:, None]
    W1t = W_in1[1][:, None]
    b1 = b_in1[:, None]
    W2T = W_in2.T.astype(BF16)
    b2 = b_in2[:, None]

    WruTe = enc_W_ru.reshape(ts, M, ts).transpose(2, 1, 0).reshape(
        ts, M * ts).astype(BF16)
    brue = enc_b_ru[:, None]
    WcTe = enc_W_c.reshape(ts, M, D).transpose(2, 1, 0).reshape(
        D, M * ts).astype(BF16)
    bce = enc_b_c[:, None]
    WruTd = dec_W_ru.reshape(ts, M, ts)[D:].transpose(2, 1, 0).reshape(
        ts, M * D).astype(BF16)
    brud = dec_b_ru[:, None]
    WcTd = dec_W_c.reshape(ts, M, D)[D:].transpose(2, 1, 0).reshape(
        D, M * D).astype(BF16)
    bcd = dec_b_c[:, None]

    Wo1T = W_out1.T.astype(BF16)
    bo1 = b_out1[:, None]
    w2c = W_out2[:, 0][:, None]
    bo2 = b_out2.reshape(1, 1)

    out = _dcgru_call(Xp, todp, adjp, adjTp, W1x, W1t, b1, W2T, b2,
                      WruTe, brue, WcTe, bce, WruTd, brud, WcTd, bcd,
                      Wo1T, bo1, w2c, bo2)
    return out.transpose(1, 0, 2)[:, :, :N, None]
